# baseline (speedup 1.0000x reference)
import jax
import jax.numpy as jnp
from jax.experimental import pallas as pl
from jax.experimental.pallas import tpu as pltpu

B, S, H, E, K = 4, 4096, 2048, 64, 2
T = B * S
BLK = 1024
GRID = T // BLK


SUB = 256
NSUB = BLK // SUB

_LOG2E = 1.4426950408889634
_LN2 = 0.6931471805599453


def _vexp(z):
    t = jnp.maximum(z * _LOG2E, -125.0)
    ri = (t - 0.5).astype(jnp.int32)
    u = (t - ri.astype(jnp.float32)) * _LN2
    p = 1.0 + u * (1.0 + u * (0.5 + u * (
        0.16666666666666666 + u * (0.041666666666666664 + u * (
            0.008333333333333333 + u * (0.001388888888888889 + u * (
                1.984126984126984e-4 + u * (2.48015873015873e-5 + u * (
                    2.7557319223985893e-6 + u * 2.755731922398589e-7)))))))))
    sc = jax.lax.bitcast_convert_type((ri + 127) << 23, jnp.float32)
    return p * sc


def _vrcp(d):
    y = jax.lax.bitcast_convert_type(
        jnp.int32(0x7EF311C3) - jax.lax.bitcast_convert_type(d, jnp.int32),
        jnp.float32)
    y = y * (2.0 - d * y)
    y = y * (2.0 - d * y)
    y = y * (2.0 - d * y)
    return y


def _router_kernel(x_ref, w_ref, tw_ref, ti_ref, aux_ref, acc_ref, buf_ref):
    i = pl.program_id(0)

    @pl.when(i == 0)
    def _():
        acc_ref[...] = jnp.zeros_like(acc_ref)

    w = w_ref[...]
    eidf = jax.lax.broadcasted_iota(jnp.int32, (SUB, E), 1).astype(jnp.float32)

    for c in range(NSUB):
        r = slice(c * SUB, (c + 1) * SUB)
        x = x_ref[0, r, :]
        logits = jax.lax.dot_general(
            x, w, (((1,), (1,)), ((), ())), preferred_element_type=jnp.float32
        )

        m = jnp.max(logits, axis=-1, keepdims=True)
        ex = jnp.exp(logits - m)
        denom = jnp.sum(ex, axis=-1, keepdims=True)

        e1 = jnp.max(ex, axis=-1, keepdims=True)
        a1 = jnp.min(jnp.where(ex == e1, eidf, 64.0), axis=-1, keepdims=True)
        hit1 = eidf == a1
        masked = jnp.where(hit1, -1.0, ex)
        e2 = jnp.max(masked, axis=-1, keepdims=True)
        a2 = jnp.min(jnp.where(masked == e2, eidf, 64.0), axis=-1, keepdims=True)
        hit2 = eidf == a2

        invq = 1.0 / (e1 + e2 + 1e-9 * denom)
        buf_ref[r, 0:1] = e1 * invq
        buf_ref[r, 1:2] = e2 * invq
        buf_ref[r, 2:3] = a1
        buf_ref[r, 3:4] = a2

        cnt = jnp.sum(jnp.where(hit1, 1.0, 0.0) + jnp.where(hit2, 1.0, 0.0),
                      axis=0, keepdims=True)
        psum = jnp.sum(ex * (1.0 / denom), axis=0, keepdims=True)
        acc_ref[0:1, :E] += cnt
        acc_ref[1:2, :E] += psum

    tt = buf_ref[...].T
    tw_ref[0, :, :] = tt[0:2, :]
    ti_ref[0, :, :] = tt[2:4, :].astype(jnp.int32)

    @pl.when(i == GRID - 1)
    def _():
        f = acc_ref[0:1, :E] * (1.0 / float(T * K))
        P = acc_ref[1:2, :E] * (1.0 / float(T))
        aux_ref[...] = 0.01 * E * jnp.sum(f * P, axis=1, keepdims=True)


NB = S // BLK


def kernel(hidden_states, gate_weight):
    tw, ti, aux = pl.pallas_call(
        _router_kernel,
        grid=(GRID,),
        in_specs=[
            pl.BlockSpec((1, BLK, H), lambda i: (i // NB, i % NB, 0)),
            pl.BlockSpec((E, H), lambda i: (0, 0)),
        ],
        out_specs=[
            pl.BlockSpec((1, K, BLK), lambda i: (i, 0, 0)),
            pl.BlockSpec((1, K, BLK), lambda i: (i, 0, 0)),
            pl.BlockSpec((1, 1), lambda i: (0, 0)),
        ],
        out_shape=[
            jax.ShapeDtypeStruct((GRID, K, BLK), jnp.float32),
            jax.ShapeDtypeStruct((GRID, K, BLK), jnp.int32),
            jax.ShapeDtypeStruct((1, 1), jnp.float32),
        ],
        scratch_shapes=[pltpu.VMEM((8, 128), jnp.float32),
                        pltpu.VMEM((BLK, 4), jnp.float32)],
    )(hidden_states, gate_weight)
    return (
        tw.transpose(0, 2, 1).reshape(B, S, K),
        ti.transpose(0, 2, 1).reshape(B, S, K).astype(jnp.int64),
        aux.reshape(()),
    )


# speedup vs baseline: 1.0485x; 1.0485x over previous
import jax
import jax.numpy as jnp
from jax.experimental import pallas as pl
from jax.experimental.pallas import tpu as pltpu

B, S, H, E, K = 4, 4096, 2048, 64, 2
T = B * S
BLK = 2048
GRID = T // BLK


SUB = 256
NSUB = BLK // SUB

def _router_kernel(x_ref, w_ref, tw_ref, ti_ref, aux_ref, acc_ref, buf_ref):
    i = pl.program_id(0)

    @pl.when(i == 0)
    def _():
        acc_ref[...] = jnp.zeros_like(acc_ref)

    w = w_ref[...]
    eidf = jax.lax.broadcasted_iota(jnp.int32, (SUB, E), 1).astype(jnp.float32)

    for c in range(NSUB):
        r = slice(c * SUB, (c + 1) * SUB)
        x = x_ref[0, r, :]
        logits = jax.lax.dot_general(
            x, w, (((1,), (1,)), ((), ())), preferred_element_type=jnp.float32
        )

        m = jnp.max(logits, axis=-1, keepdims=True)
        ex = jnp.exp(logits - m)
        denom = jnp.sum(ex, axis=-1, keepdims=True)

        e1 = jnp.max(ex, axis=-1, keepdims=True)
        a1 = jnp.min(jnp.where(ex == e1, eidf, 64.0), axis=-1, keepdims=True)
        hit1 = eidf == a1
        masked = jnp.where(hit1, -1.0, ex)
        e2 = jnp.max(masked, axis=-1, keepdims=True)
        a2 = jnp.min(jnp.where(masked == e2, eidf, 64.0), axis=-1, keepdims=True)
        hit2 = eidf == a2

        invq = 1.0 / (e1 + e2 + 1e-9 * denom)
        buf_ref[r, 0:1] = e1 * invq
        buf_ref[r, 1:2] = e2 * invq
        buf_ref[r, 2:3] = a1
        buf_ref[r, 3:4] = a2

        cnt = jnp.sum(jnp.where(hit1, 1.0, 0.0) + jnp.where(hit2, 1.0, 0.0),
                      axis=0, keepdims=True)
        psum = jnp.sum(ex * (1.0 / denom), axis=0, keepdims=True)
        acc_ref[0:1, :E] += cnt
        acc_ref[1:2, :E] += psum

    tt = buf_ref[...].T
    tw_ref[0, :, :] = tt[0:2, :]
    ti_ref[0, :, :] = tt[2:4, :].astype(jnp.int32)

    @pl.when(i == GRID - 1)
    def _():
        f = acc_ref[0:1, :E] * (1.0 / float(T * K))
        P = acc_ref[1:2, :E] * (1.0 / float(T))
        aux_ref[...] = 0.01 * E * jnp.sum(f * P, axis=1, keepdims=True)


NB = S // BLK


def kernel(hidden_states, gate_weight):
    tw, ti, aux = pl.pallas_call(
        _router_kernel,
        grid=(GRID,),
        in_specs=[
            pl.BlockSpec((1, BLK, H), lambda i: (i // NB, i % NB, 0)),
            pl.BlockSpec((E, H), lambda i: (0, 0)),
        ],
        out_specs=[
            pl.BlockSpec((1, K, BLK), lambda i: (i, 0, 0)),
            pl.BlockSpec((1, K, BLK), lambda i: (i, 0, 0)),
            pl.BlockSpec((1, 1), lambda i: (0, 0)),
        ],
        out_shape=[
            jax.ShapeDtypeStruct((GRID, K, BLK), jnp.float32),
            jax.ShapeDtypeStruct((GRID, K, BLK), jnp.int32),
            jax.ShapeDtypeStruct((1, 1), jnp.float32),
        ],
        scratch_shapes=[pltpu.VMEM((8, 128), jnp.float32),
                        pltpu.VMEM((BLK, 4), jnp.float32)],
    )(hidden_states, gate_weight)
    return (
        tw.transpose(0, 2, 1).reshape(B, S, K),
        ti.transpose(0, 2, 1).reshape(B, S, K).astype(jnp.int64),
        aux.reshape(()),
    )
